# Initial kernel scaffold; baseline (speedup 1.0000x reference)
import jax
import jax.numpy as jnp
from jax.experimental import pallas as pl


def kernel(image_feature, point_cloud_feature, prev_image_feature, prev_point_cloud_feature, rel_dist_mask, prev_spatial, img_w1, img_b1, img_w2, img_b2, img_w3, img_b3, img_ln_g, img_ln_b, pc_w1, pc_b1, pc_w2, pc_b2, pc_w3, pc_b3, pc_ln_g, pc_ln_b, fu_w1, fu_b1, fu_w2, fu_b2, fu_ln_g, fu_ln_b):
    raise NotImplementedError("write your pallas kernel here")


# speedup vs baseline: 10.0340x; 10.0340x over previous
import jax
import jax.numpy as jnp
from jax import lax
from jax.experimental import pallas as pl
from jax.experimental.pallas import tpu as pltpu

_B, _N = 16, 256
_IMG_D, _PC_D = 2048, 768
_VIS_D, _SP_D = 768, 8
_THR = 0.5
_EPS = 1e-8
_NEG = -1e30
_F32 = jnp.float32


def _sim_body(img_ref, pc_ref, pimg_ref, ppc_ref, mask_ref, s_ref):
    img = img_ref[0]
    pc = pc_ref[0]
    pimg = pimg_ref[0]
    ppc = ppc_ref[0]
    dn = (((1,), (1,)), ((), ()))
    dot = lax.dot_general(img, pimg, dn, preferred_element_type=_F32,
                          precision=lax.Precision.HIGHEST)
    dot += lax.dot_general(pc, ppc, dn, preferred_element_type=_F32,
                           precision=lax.Precision.HIGHEST)
    nf = jnp.maximum(jnp.sqrt(jnp.sum(img * img, 1, keepdims=True)
                              + jnp.sum(pc * pc, 1, keepdims=True)), _EPS)
    npr = jnp.maximum(jnp.sqrt(jnp.sum(pimg * pimg, 1, keepdims=True)
                               + jnp.sum(ppc * ppc, 1, keepdims=True)), _EPS)
    sim = dot / (nf * npr.reshape(1, _N))
    s_ref[:, 0, 0, :] = jnp.where(mask_ref[0] != 0, sim, _NEG)


def _match_body(s_ref, p_ref):
    def step(i, visited):
        row = s_ref[i, :, 0, :]
        s = jnp.where(visited != 0, _NEG, row)
        m = jnp.max(s, axis=1, keepdims=True)
        iota = lax.broadcasted_iota(jnp.int32, (_B, _N), 1)
        cand = jnp.where(s == m, iota, _N)
        j = jnp.min(cand, axis=1, keepdims=True)
        hit = (iota == j) & (m >= _THR)
        p_ref[i, :, 0, :] = hit.astype(_F32)
        return visited | hit.astype(jnp.int32)

    lax.fori_loop(0, _N, step, jnp.zeros((_B, _N), jnp.int32))


def _layer_norm(x, g, b):
    m = jnp.mean(x, axis=1, keepdims=True)
    v = jnp.mean((x - m) ** 2, axis=1, keepdims=True)
    return (x - m) / jnp.sqrt(v + 1e-5) * g + b


def _ordered_body(p_ref, pimg_ref, ppc_ref, psp_ref, oimg_ref, opc_ref,
                  sp_ref):
    pmat = p_ref[:, 0, 0, :]
    hp = lax.Precision.HIGHEST
    oimg_ref[0] = jnp.dot(pmat, pimg_ref[0], preferred_element_type=_F32,
                          precision=hp)
    opc_ref[0] = jnp.dot(pmat, ppc_ref[0], preferred_element_type=_F32,
                         precision=hp)
    sp_ref[0] = jnp.dot(pmat, psp_ref[0], preferred_element_type=_F32,
                        precision=hp)


def _img_l1_body(img_ref, oimg_ref, w1a_ref, w1b_ref, b1_ref, out_ref):
    h = (jnp.dot(img_ref[0], w1a_ref[...], preferred_element_type=_F32)
         + jnp.dot(oimg_ref[0], w1b_ref[...], preferred_element_type=_F32)
         + b1_ref[...])
    out_ref[0] = jnp.maximum(h, 0.0)


def _img_l23_body(x_ref, w2_ref, b2_ref, w3_ref, b3_ref, g_ref, b_ref,
                  out_ref):
    h = jnp.dot(x_ref[0], w2_ref[...], preferred_element_type=_F32) + b2_ref[...]
    h = jnp.dot(h, w3_ref[...], preferred_element_type=_F32) + b3_ref[...]
    out_ref[0] = _layer_norm(h, g_ref[...], b_ref[...])


def _pc_body(pc_ref, opc_ref, w1a_ref, w1b_ref, b1_ref, w2_ref, b2_ref,
             w3_ref, b3_ref, g_ref, b_ref, out_ref):
    h = (jnp.dot(pc_ref[0], w1a_ref[...], preferred_element_type=_F32)
         + jnp.dot(opc_ref[0], w1b_ref[...], preferred_element_type=_F32)
         + b1_ref[...])
    h = jnp.maximum(h, 0.0)
    h = jnp.dot(h, w2_ref[...], preferred_element_type=_F32) + b2_ref[...]
    h = jnp.dot(h, w3_ref[...], preferred_element_type=_F32) + b3_ref[...]
    out_ref[0] = _layer_norm(h, g_ref[...], b_ref[...])


def _fu_body(hi_ref, hp_ref, w1a_ref, w1b_ref, b1_ref, w2_ref, b2_ref,
             g_ref, b_ref, vis_ref):
    h = (jnp.dot(hi_ref[0], w1a_ref[...], preferred_element_type=_F32)
         + jnp.dot(hp_ref[0], w1b_ref[...], preferred_element_type=_F32)
         + b1_ref[...])
    h = jnp.maximum(h, 0.0)
    h = jnp.dot(h, w2_ref[...], preferred_element_type=_F32) + b2_ref[...]
    vis_ref[0] = _layer_norm(h, g_ref[...], b_ref[...])


def kernel(image_feature, point_cloud_feature, prev_image_feature,
           prev_point_cloud_feature, rel_dist_mask, prev_spatial,
           img_w1, img_b1, img_w2, img_b2, img_w3, img_b3, img_ln_g, img_ln_b,
           pc_w1, pc_b1, pc_w2, pc_b2, pc_w3, pc_b3, pc_ln_g, pc_ln_b,
           fu_w1, fu_b1, fu_w2, fu_b2, fu_ln_g, fu_ln_b):
    maskf = rel_dist_mask.astype(_F32)

    s_t = pl.pallas_call(
        _sim_body,
        grid=(_B,),
        in_specs=[
            pl.BlockSpec((1, _N, _IMG_D), lambda b: (b, 0, 0)),
            pl.BlockSpec((1, _N, _PC_D), lambda b: (b, 0, 0)),
            pl.BlockSpec((1, _N, _IMG_D), lambda b: (b, 0, 0)),
            pl.BlockSpec((1, _N, _PC_D), lambda b: (b, 0, 0)),
            pl.BlockSpec((1, _N, _N), lambda b: (b, 0, 0)),
        ],
        out_specs=pl.BlockSpec((_N, 1, 1, _N), lambda b: (0, b, 0, 0)),
        out_shape=jax.ShapeDtypeStruct((_N, _B, 1, _N), _F32),
    )(image_feature, point_cloud_feature, prev_image_feature,
      prev_point_cloud_feature, maskf)

    p_t = pl.pallas_call(
        _match_body,
        out_shape=jax.ShapeDtypeStruct((_N, _B, 1, _N), _F32),
    )(s_t)

    full = lambda a: pl.BlockSpec(a.shape, lambda b: (0,) * a.ndim)
    bat = lambda d: pl.BlockSpec((1, _N, d), lambda b: (b, 0, 0))
    iw1a, iw1b = img_w1[:_IMG_D], img_w1[_IMG_D:]
    pw1a, pw1b = pc_w1[:_PC_D], pc_w1[_PC_D:]
    fw1a, fw1b = fu_w1[:_VIS_D], fu_w1[_VIS_D:]
    row = lambda a: a.reshape(1, -1)
    ib1, ib2, ib3 = row(img_b1), row(img_b2), row(img_b3)
    ilg, ilb = row(img_ln_g), row(img_ln_b)
    pb1, pb2, pb3 = row(pc_b1), row(pc_b2), row(pc_b3)
    plg, plb = row(pc_ln_g), row(pc_ln_b)
    fb1, fb2 = row(fu_b1), row(fu_b2)
    flg, flb = row(fu_ln_g), row(fu_ln_b)

    oimg, opc, new_sp = pl.pallas_call(
        _ordered_body,
        grid=(_B,),
        in_specs=[
            pl.BlockSpec((_N, 1, 1, _N), lambda b: (0, b, 0, 0)),
            bat(_IMG_D), bat(_PC_D), bat(_SP_D),
        ],
        out_specs=[bat(_IMG_D), bat(_PC_D), bat(_SP_D)],
        out_shape=[
            jax.ShapeDtypeStruct((_B, _N, _IMG_D), _F32),
            jax.ShapeDtypeStruct((_B, _N, _PC_D), _F32),
            jax.ShapeDtypeStruct((_B, _N, _SP_D), _F32),
        ],
    )(p_t, prev_image_feature, prev_point_cloud_feature, prev_spatial)

    x1 = pl.pallas_call(
        _img_l1_body,
        grid=(_B,),
        in_specs=[bat(_IMG_D), bat(_IMG_D), full(iw1a), full(iw1b), full(ib1)],
        out_specs=bat(_IMG_D),
        out_shape=jax.ShapeDtypeStruct((_B, _N, _IMG_D), _F32),
    )(image_feature, oimg, iw1a, iw1b, ib1)

    hi = pl.pallas_call(
        _img_l23_body,
        grid=(_B,),
        in_specs=[bat(_IMG_D), full(img_w2), full(ib2), full(img_w3),
                  full(ib3), full(ilg), full(ilb)],
        out_specs=bat(_VIS_D),
        out_shape=jax.ShapeDtypeStruct((_B, _N, _VIS_D), _F32),
    )(x1, img_w2, ib2, img_w3, ib3, ilg, ilb)

    hpc = pl.pallas_call(
        _pc_body,
        grid=(_B,),
        in_specs=[bat(_PC_D), bat(_PC_D), full(pw1a), full(pw1b), full(pb1),
                  full(pc_w2), full(pb2), full(pc_w3), full(pb3), full(plg),
                  full(plb)],
        out_specs=bat(_VIS_D),
        out_shape=jax.ShapeDtypeStruct((_B, _N, _VIS_D), _F32),
    )(point_cloud_feature, opc, pw1a, pw1b, pb1, pc_w2, pb2, pc_w3, pb3,
      plg, plb)

    vis = pl.pallas_call(
        _fu_body,
        grid=(_B,),
        in_specs=[bat(_VIS_D), bat(_VIS_D), full(fw1a), full(fw1b), full(fb1),
                  full(fu_w2), full(fb2), full(flg), full(flb)],
        out_specs=bat(_VIS_D),
        out_shape=jax.ShapeDtypeStruct((_B, _N, _VIS_D), _F32),
    )(hi, hpc, fw1a, fw1b, fb1, fu_w2, fb2, flg, flb)

    return vis, new_sp


# speedup vs baseline: 10.4265x; 1.0391x over previous
import jax
import jax.numpy as jnp
from jax import lax
from jax.experimental import pallas as pl
from jax.experimental.pallas import tpu as pltpu

_B, _N = 16, 256
_IMG_D, _PC_D = 2048, 768
_VIS_D, _SP_D = 768, 8
_THR = 0.5
_EPS = 1e-8
_NEG = -1e30
_F32 = jnp.float32
_BF16 = jnp.bfloat16


def _sim_body(img_ref, pc_ref, pimg_ref, ppc_ref, mask_ref, s_ref):
    img = img_ref[0]
    pc = pc_ref[0]
    pimg = pimg_ref[0]
    ppc = ppc_ref[0]
    dn = (((1,), (1,)), ((), ()))
    dot = lax.dot_general(img, pimg, dn, preferred_element_type=_F32)
    dot += lax.dot_general(pc, ppc, dn, preferred_element_type=_F32)
    sq = lambda a: jnp.sum(a.astype(_F32) ** 2, 1, keepdims=True)
    nf = jnp.maximum(jnp.sqrt(sq(img) + sq(pc)), _EPS)
    npr = jnp.maximum(jnp.sqrt(sq(pimg) + sq(ppc)), _EPS)
    sim = dot / (nf * npr.reshape(1, _N))
    s_ref[:, 0, 0, :] = jnp.where(mask_ref[0] != 0, sim, _NEG)


def _match_body(s_ref, p_ref):
    def step(i, visited):
        row = s_ref[i, :, 0, :]
        s = jnp.where(visited != 0, _NEG, row)
        m = jnp.max(s, axis=1, keepdims=True)
        iota = lax.broadcasted_iota(jnp.int32, (_B, _N), 1)
        cand = jnp.where(s == m, iota, _N)
        j = jnp.min(cand, axis=1, keepdims=True)
        hit = (iota == j) & (m >= _THR)
        p_ref[i, :, 0, :] = hit.astype(_F32)
        return visited | hit.astype(jnp.int32)

    lax.fori_loop(0, _N, step, jnp.zeros((_B, _N), jnp.int32))


def _layer_norm(x, g, b):
    m = jnp.mean(x, axis=1, keepdims=True)
    v = jnp.mean((x - m) ** 2, axis=1, keepdims=True)
    return (x - m) / jnp.sqrt(v + 1e-5) * g + b


def _img_chain_body(p_ref, img_ref, pimg_ref, w1a_ref, w1b_ref, b1_ref,
                    w2_ref, b2_ref, w3_ref, b3_ref, g_ref, bb_ref, hi_ref):
    pmat = p_ref[:, 0, 0, :].astype(_BF16)
    oimg = jnp.dot(pmat, pimg_ref[0], preferred_element_type=_F32)
    h = (jnp.dot(img_ref[0], w1a_ref[...], preferred_element_type=_F32)
         + jnp.dot(oimg.astype(_BF16), w1b_ref[...], preferred_element_type=_F32)
         + b1_ref[...])
    h = jnp.maximum(h, 0.0).astype(_BF16)
    h = jnp.dot(h, w2_ref[...], preferred_element_type=_F32) + b2_ref[...]
    h = jnp.dot(h.astype(_BF16), w3_ref[...], preferred_element_type=_F32)
    h = h + b3_ref[...]
    hi_ref[0] = _layer_norm(h, g_ref[...], bb_ref[...])


def _pcfu_body(p_ref, pc_ref, ppc_ref, psp_ref, hi_ref,
               pw1a_ref, pw1b_ref, pb1_ref, pw2_ref, pb2_ref, pw3_ref,
               pb3_ref, plg_ref, plb_ref,
               fw1a_ref, fw1b_ref, fb1_ref, fw2_ref, fb2_ref, flg_ref,
               flb_ref, vis_ref, sp_ref):
    pmatf = p_ref[:, 0, 0, :]
    pmat = pmatf.astype(_BF16)
    sp_ref[0] = jnp.dot(pmatf, psp_ref[0], preferred_element_type=_F32)
    opc = jnp.dot(pmat, ppc_ref[0], preferred_element_type=_F32)
    h = (jnp.dot(pc_ref[0], pw1a_ref[...], preferred_element_type=_F32)
         + jnp.dot(opc.astype(_BF16), pw1b_ref[...], preferred_element_type=_F32)
         + pb1_ref[...])
    h = jnp.maximum(h, 0.0).astype(_BF16)
    h = jnp.dot(h, pw2_ref[...], preferred_element_type=_F32) + pb2_ref[...]
    h = jnp.dot(h.astype(_BF16), pw3_ref[...], preferred_element_type=_F32)
    h = h + pb3_ref[...]
    hp = _layer_norm(h, plg_ref[...], plb_ref[...])

    h = (jnp.dot(hi_ref[0].astype(_BF16), fw1a_ref[...],
                 preferred_element_type=_F32)
         + jnp.dot(hp.astype(_BF16), fw1b_ref[...], preferred_element_type=_F32)
         + fb1_ref[...])
    h = jnp.maximum(h, 0.0).astype(_BF16)
    h = jnp.dot(h, fw2_ref[...], preferred_element_type=_F32) + fb2_ref[...]
    vis_ref[0] = _layer_norm(h, flg_ref[...], flb_ref[...])


def kernel(image_feature, point_cloud_feature, prev_image_feature,
           prev_point_cloud_feature, rel_dist_mask, prev_spatial,
           img_w1, img_b1, img_w2, img_b2, img_w3, img_b3, img_ln_g, img_ln_b,
           pc_w1, pc_b1, pc_w2, pc_b2, pc_w3, pc_b3, pc_ln_g, pc_ln_b,
           fu_w1, fu_b1, fu_w2, fu_b2, fu_ln_g, fu_ln_b):
    maskf = rel_dist_mask.astype(_F32)

    s_t = pl.pallas_call(
        _sim_body,
        grid=(_B,),
        in_specs=[
            pl.BlockSpec((1, _N, _IMG_D), lambda b: (b, 0, 0)),
            pl.BlockSpec((1, _N, _PC_D), lambda b: (b, 0, 0)),
            pl.BlockSpec((1, _N, _IMG_D), lambda b: (b, 0, 0)),
            pl.BlockSpec((1, _N, _PC_D), lambda b: (b, 0, 0)),
            pl.BlockSpec((1, _N, _N), lambda b: (b, 0, 0)),
        ],
        out_specs=pl.BlockSpec((_N, 1, 1, _N), lambda b: (0, b, 0, 0)),
        out_shape=jax.ShapeDtypeStruct((_N, _B, 1, _N), _F32),
    )(image_feature, point_cloud_feature, prev_image_feature,
      prev_point_cloud_feature, maskf)

    p_t = pl.pallas_call(
        _match_body,
        out_shape=jax.ShapeDtypeStruct((_N, _B, 1, _N), _F32),
    )(s_t)

    full = lambda a: pl.BlockSpec(a.shape, lambda b: (0,) * a.ndim)
    bat = lambda d: pl.BlockSpec((1, _N, d), lambda b: (b, 0, 0))
    pspec = pl.BlockSpec((_N, 1, 1, _N), lambda b: (0, b, 0, 0))
    h = lambda a: a.astype(_BF16)
    row = lambda a: a.reshape(1, -1)
    iw1a, iw1b = h(img_w1[:_IMG_D]), h(img_w1[_IMG_D:])
    pw1a, pw1b = h(pc_w1[:_PC_D]), h(pc_w1[_PC_D:])
    fw1a, fw1b = h(fu_w1[:_VIS_D]), h(fu_w1[_VIS_D:])
    iw2, iw3 = h(img_w2), h(img_w3)
    pw2, pw3 = h(pc_w2), h(pc_w3)
    fw2 = h(fu_w2)
    img_h, pc_h = h(image_feature), h(point_cloud_feature)
    pimg_h, ppc_h = h(prev_image_feature), h(prev_point_cloud_feature)
    ib1, ib2, ib3 = row(img_b1), row(img_b2), row(img_b3)
    ilg, ilb = row(img_ln_g), row(img_ln_b)
    pb1, pb2, pb3 = row(pc_b1), row(pc_b2), row(pc_b3)
    plg, plb = row(pc_ln_g), row(pc_ln_b)
    fb1, fb2 = row(fu_b1), row(fu_b2)
    flg, flb = row(fu_ln_g), row(fu_ln_b)

    hi = pl.pallas_call(
        _img_chain_body,
        grid=(_B,),
        in_specs=[pspec, bat(_IMG_D), bat(_IMG_D), full(iw1a), full(iw1b),
                  full(ib1), full(iw2), full(ib2), full(iw3), full(ib3),
                  full(ilg), full(ilb)],
        out_specs=bat(_VIS_D),
        out_shape=jax.ShapeDtypeStruct((_B, _N, _VIS_D), _F32),
    )(p_t, img_h, pimg_h, iw1a, iw1b, ib1, iw2, ib2, iw3, ib3, ilg, ilb)

    vis, new_sp = pl.pallas_call(
        _pcfu_body,
        grid=(_B,),
        in_specs=[pspec, bat(_PC_D), bat(_PC_D), bat(_SP_D), bat(_VIS_D),
                  full(pw1a), full(pw1b), full(pb1), full(pw2), full(pb2),
                  full(pw3), full(pb3), full(plg), full(plb),
                  full(fw1a), full(fw1b), full(fb1), full(fw2), full(fb2),
                  full(flg), full(flb)],
        out_specs=[bat(_VIS_D), bat(_SP_D)],
        out_shape=[
            jax.ShapeDtypeStruct((_B, _N, _VIS_D), _F32),
            jax.ShapeDtypeStruct((_B, _N, _SP_D), _F32),
        ],
    )(p_t, pc_h, ppc_h, prev_spatial, hi,
      pw1a, pw1b, pb1, pw2, pb2, pw3, pb3, plg, plb,
      fw1a, fw1b, fb1, fw2, fb2, flg, flb)

    return vis, new_sp


# speedup vs baseline: 12.1379x; 1.1641x over previous
import jax
import jax.numpy as jnp
from jax import lax
from jax.experimental import pallas as pl
from jax.experimental.pallas import tpu as pltpu

_B, _N = 16, 256
_IMG_D, _PC_D = 2048, 768
_VIS_D, _SP_D = 768, 8
_THR = 0.5
_EPS = 1e-8
_NEG = -1e30
_F32 = jnp.float32
_BF16 = jnp.bfloat16


def _sim_body(img_ref, pc_ref, pimg_ref, ppc_ref, mask_ref, s_ref):
    img = img_ref[0]
    pc = pc_ref[0]
    pimg = pimg_ref[0]
    ppc = ppc_ref[0]
    dn = (((1,), (1,)), ((), ()))
    dot = lax.dot_general(img, pimg, dn, preferred_element_type=_F32)
    dot += lax.dot_general(pc, ppc, dn, preferred_element_type=_F32)
    sq = lambda a: jnp.sum(a.astype(_F32) ** 2, 1, keepdims=True)
    nf = jnp.maximum(jnp.sqrt(sq(img) + sq(pc)), _EPS)
    npr = jnp.maximum(jnp.sqrt(sq(pimg) + sq(ppc)), _EPS)
    sim = dot / (nf * npr.reshape(1, _N))
    s_ref[:, 0, 0, :] = jnp.where(mask_ref[0] != 0, sim, _NEG)


def _match_body(s_ref, p_ref):
    def step(i, visited):
        row = s_ref[i, :, 0, :]
        s = jnp.where(visited != 0, _NEG, row)
        m = jnp.max(s, axis=1, keepdims=True)
        iota = lax.broadcasted_iota(jnp.int32, (_B, _N), 1)
        cand = jnp.where(s == m, iota, _N)
        j = jnp.min(cand, axis=1, keepdims=True)
        hit = (iota == j) & (m >= _THR)
        p_ref[i, :, 0, :] = hit.astype(_F32)
        return visited | hit.astype(jnp.int32)

    def slow():
        lax.fori_loop(0, _N, step, jnp.zeros((_B, _N), jnp.int32))

    def fast():
        p_ref[...] = jnp.zeros((_N, _B, 1, _N), _F32)

    lax.cond(jnp.max(s_ref[...]) >= _THR, slow, fast)


def _layer_norm(x, g, b):
    m = jnp.mean(x, axis=1, keepdims=True)
    v = jnp.mean((x - m) ** 2, axis=1, keepdims=True)
    return (x - m) / jnp.sqrt(v + 1e-5) * g + b


def _img_chain_body(p_ref, img_ref, pimg_ref, w1a_ref, w1b_ref, b1_ref,
                    w2_ref, b2_ref, w3_ref, b3_ref, g_ref, bb_ref, hi_ref):
    pmat = p_ref[:, 0, 0, :].astype(_BF16)
    oimg = jnp.dot(pmat, pimg_ref[0], preferred_element_type=_F32)
    h = (jnp.dot(img_ref[0], w1a_ref[...], preferred_element_type=_F32)
         + jnp.dot(oimg.astype(_BF16), w1b_ref[...], preferred_element_type=_F32)
         + b1_ref[...])
    h = jnp.maximum(h, 0.0).astype(_BF16)
    h = jnp.dot(h, w2_ref[...], preferred_element_type=_F32) + b2_ref[...]
    h = jnp.dot(h.astype(_BF16), w3_ref[...], preferred_element_type=_F32)
    h = h + b3_ref[...]
    hi_ref[0] = _layer_norm(h, g_ref[...], bb_ref[...])


def _pcfu_body(p_ref, pc_ref, ppc_ref, psp_ref, hi_ref,
               pw1a_ref, pw1b_ref, pb1_ref, pw2_ref, pb2_ref, pw3_ref,
               pb3_ref, plg_ref, plb_ref,
               fw1a_ref, fw1b_ref, fb1_ref, fw2_ref, fb2_ref, flg_ref,
               flb_ref, vis_ref, sp_ref):
    pmatf = p_ref[:, 0, 0, :]
    pmat = pmatf.astype(_BF16)
    sp_ref[0] = jnp.dot(pmatf, psp_ref[0], preferred_element_type=_F32)
    opc = jnp.dot(pmat, ppc_ref[0], preferred_element_type=_F32)
    h = (jnp.dot(pc_ref[0], pw1a_ref[...], preferred_element_type=_F32)
         + jnp.dot(opc.astype(_BF16), pw1b_ref[...], preferred_element_type=_F32)
         + pb1_ref[...])
    h = jnp.maximum(h, 0.0).astype(_BF16)
    h = jnp.dot(h, pw2_ref[...], preferred_element_type=_F32) + pb2_ref[...]
    h = jnp.dot(h.astype(_BF16), pw3_ref[...], preferred_element_type=_F32)
    h = h + pb3_ref[...]
    hp = _layer_norm(h, plg_ref[...], plb_ref[...])

    h = (jnp.dot(hi_ref[0].astype(_BF16), fw1a_ref[...],
                 preferred_element_type=_F32)
         + jnp.dot(hp.astype(_BF16), fw1b_ref[...], preferred_element_type=_F32)
         + fb1_ref[...])
    h = jnp.maximum(h, 0.0).astype(_BF16)
    h = jnp.dot(h, fw2_ref[...], preferred_element_type=_F32) + fb2_ref[...]
    vis_ref[0] = _layer_norm(h, flg_ref[...], flb_ref[...])


def kernel(image_feature, point_cloud_feature, prev_image_feature,
           prev_point_cloud_feature, rel_dist_mask, prev_spatial,
           img_w1, img_b1, img_w2, img_b2, img_w3, img_b3, img_ln_g, img_ln_b,
           pc_w1, pc_b1, pc_w2, pc_b2, pc_w3, pc_b3, pc_ln_g, pc_ln_b,
           fu_w1, fu_b1, fu_w2, fu_b2, fu_ln_g, fu_ln_b):
    maskf = rel_dist_mask.astype(_F32)

    s_t = pl.pallas_call(
        _sim_body,
        grid=(_B,),
        in_specs=[
            pl.BlockSpec((1, _N, _IMG_D), lambda b: (b, 0, 0)),
            pl.BlockSpec((1, _N, _PC_D), lambda b: (b, 0, 0)),
            pl.BlockSpec((1, _N, _IMG_D), lambda b: (b, 0, 0)),
            pl.BlockSpec((1, _N, _PC_D), lambda b: (b, 0, 0)),
            pl.BlockSpec((1, _N, _N), lambda b: (b, 0, 0)),
        ],
        out_specs=pl.BlockSpec((_N, 1, 1, _N), lambda b: (0, b, 0, 0)),
        out_shape=jax.ShapeDtypeStruct((_N, _B, 1, _N), _F32),
    )(image_feature, point_cloud_feature, prev_image_feature,
      prev_point_cloud_feature, maskf)

    p_t = pl.pallas_call(
        _match_body,
        out_shape=jax.ShapeDtypeStruct((_N, _B, 1, _N), _F32),
    )(s_t)

    full = lambda a: pl.BlockSpec(a.shape, lambda b: (0,) * a.ndim)
    bat = lambda d: pl.BlockSpec((1, _N, d), lambda b: (b, 0, 0))
    pspec = pl.BlockSpec((_N, 1, 1, _N), lambda b: (0, b, 0, 0))
    h = lambda a: a.astype(_BF16)
    row = lambda a: a.reshape(1, -1)
    iw1a, iw1b = h(img_w1[:_IMG_D]), h(img_w1[_IMG_D:])
    pw1a, pw1b = h(pc_w1[:_PC_D]), h(pc_w1[_PC_D:])
    fw1a, fw1b = h(fu_w1[:_VIS_D]), h(fu_w1[_VIS_D:])
    iw2, iw3 = h(img_w2), h(img_w3)
    pw2, pw3 = h(pc_w2), h(pc_w3)
    fw2 = h(fu_w2)
    img_h, pc_h = h(image_feature), h(point_cloud_feature)
    pimg_h, ppc_h = h(prev_image_feature), h(prev_point_cloud_feature)
    ib1, ib2, ib3 = row(img_b1), row(img_b2), row(img_b3)
    ilg, ilb = row(img_ln_g), row(img_ln_b)
    pb1, pb2, pb3 = row(pc_b1), row(pc_b2), row(pc_b3)
    plg, plb = row(pc_ln_g), row(pc_ln_b)
    fb1, fb2 = row(fu_b1), row(fu_b2)
    flg, flb = row(fu_ln_g), row(fu_ln_b)

    hi = pl.pallas_call(
        _img_chain_body,
        grid=(_B,),
        in_specs=[pspec, bat(_IMG_D), bat(_IMG_D), full(iw1a), full(iw1b),
                  full(ib1), full(iw2), full(ib2), full(iw3), full(ib3),
                  full(ilg), full(ilb)],
        out_specs=bat(_VIS_D),
        out_shape=jax.ShapeDtypeStruct((_B, _N, _VIS_D), _F32),
    )(p_t, img_h, pimg_h, iw1a, iw1b, ib1, iw2, ib2, iw3, ib3, ilg, ilb)

    vis, new_sp = pl.pallas_call(
        _pcfu_body,
        grid=(_B,),
        in_specs=[pspec, bat(_PC_D), bat(_PC_D), bat(_SP_D), bat(_VIS_D),
                  full(pw1a), full(pw1b), full(pb1), full(pw2), full(pb2),
                  full(pw3), full(pb3), full(plg), full(plb),
                  full(fw1a), full(fw1b), full(fb1), full(fw2), full(fb2),
                  full(flg), full(flb)],
        out_specs=[bat(_VIS_D), bat(_SP_D)],
        out_shape=[
            jax.ShapeDtypeStruct((_B, _N, _VIS_D), _F32),
            jax.ShapeDtypeStruct((_B, _N, _SP_D), _F32),
        ],
    )(p_t, pc_h, ppc_h, prev_spatial, hi,
      pw1a, pw1b, pb1, pw2, pb2, pw3, pb3, plg, plb,
      fw1a, fw1b, fb1, fw2, fb2, flg, flb)

    return vis, new_sp


# speedup vs baseline: 12.2969x; 1.0131x over previous
import jax
import jax.numpy as jnp
from jax import lax
from jax.experimental import pallas as pl
from jax.experimental.pallas import tpu as pltpu

_B, _N = 16, 256
_IMG_D, _PC_D = 2048, 768
_VIS_D, _SP_D = 768, 8
_THR = 0.5
_EPS = 1e-8
_NEG = -1e30
_F32 = jnp.float32
_BF16 = jnp.bfloat16


def _sim_body(img_ref, pc_ref, pimg_ref, ppc_ref, mask_ref, s_ref):
    img = img_ref[0]
    pc = pc_ref[0]
    pimg = pimg_ref[0]
    ppc = ppc_ref[0]
    dn = (((1,), (1,)), ((), ()))
    dot = lax.dot_general(img, pimg, dn, preferred_element_type=_F32)
    dot += lax.dot_general(pc, ppc, dn, preferred_element_type=_F32)
    sq = lambda a: jnp.sum(a.astype(_F32) ** 2, 1, keepdims=True)
    nf = jnp.maximum(jnp.sqrt(sq(img) + sq(pc)), _EPS)
    npr = jnp.maximum(jnp.sqrt(sq(pimg) + sq(ppc)), _EPS)
    sim = dot / (nf * npr.reshape(1, _N))
    s_ref[:, 0, 0, :] = jnp.where(mask_ref[0] != 0, sim, _NEG)


def _match_body(s_ref, p_ref):
    def step(i, visited):
        row = s_ref[i, :, 0, :]
        s = jnp.where(visited != 0, _NEG, row)
        m = jnp.max(s, axis=1, keepdims=True)
        iota = lax.broadcasted_iota(jnp.int32, (_B, _N), 1)
        cand = jnp.where(s == m, iota, _N)
        j = jnp.min(cand, axis=1, keepdims=True)
        hit = (iota == j) & (m >= _THR)
        p_ref[i, :, 0, :] = hit.astype(_F32)
        return visited | hit.astype(jnp.int32)

    def slow():
        lax.fori_loop(0, _N, step, jnp.zeros((_B, _N), jnp.int32))

    def fast():
        p_ref[...] = jnp.zeros((_N, _B, 1, _N), _F32)

    lax.cond(jnp.max(s_ref[...]) >= _THR, slow, fast)


def _layer_norm(x, g, b):
    m = jnp.mean(x, axis=1, keepdims=True)
    v = jnp.mean((x - m) ** 2, axis=1, keepdims=True)
    return (x - m) / jnp.sqrt(v + 1e-5) * g + b


def _img_chain_body(p_ref, img_ref, pimg_ref, w1a_hbm, w1b_hbm, w2_hbm,
                    w3_hbm, b1_ref, b2_ref, b3_ref, g_ref, bb_ref, hi_ref,
                    w1a_ref, w1b_ref, w2_ref, w3_ref, sem):
    @pl.when(pl.program_id(0) == 0)
    def _load_weights():
        cps = [pltpu.make_async_copy(w1a_hbm, w1a_ref, sem),
               pltpu.make_async_copy(w1b_hbm, w1b_ref, sem),
               pltpu.make_async_copy(w2_hbm, w2_ref, sem),
               pltpu.make_async_copy(w3_hbm, w3_ref, sem)]
        for c in cps:
            c.start()
        for c in cps:
            c.wait()

    pmat = p_ref[:, 0, 0, :].astype(_BF16)
    oimg = jnp.dot(pmat, pimg_ref[0], preferred_element_type=_F32)
    h = (jnp.dot(img_ref[0], w1a_ref[...], preferred_element_type=_F32)
         + jnp.dot(oimg.astype(_BF16), w1b_ref[...], preferred_element_type=_F32)
         + b1_ref[...])
    h = jnp.maximum(h, 0.0).astype(_BF16)
    h = jnp.dot(h, w2_ref[...], preferred_element_type=_F32) + b2_ref[...]
    h = jnp.dot(h.astype(_BF16), w3_ref[...], preferred_element_type=_F32)
    h = h + b3_ref[...]
    hi_ref[0] = _layer_norm(h, g_ref[...], bb_ref[...])


def _pcfu_body(p_ref, pc_ref, ppc_ref, psp_ref, hi_ref,
               pw1a_hbm, pw1b_hbm, pw2_hbm, pw3_hbm,
               fw1a_hbm, fw1b_hbm, fw2_hbm,
               pb1_ref, pb2_ref, pb3_ref, plg_ref, plb_ref,
               fb1_ref, fb2_ref, flg_ref, flb_ref, vis_ref, sp_ref,
               pw1a_ref, pw1b_ref, pw2_ref, pw3_ref,
               fw1a_ref, fw1b_ref, fw2_ref, sem):
    @pl.when(pl.program_id(0) == 0)
    def _load_weights():
        cps = [pltpu.make_async_copy(pw1a_hbm, pw1a_ref, sem),
               pltpu.make_async_copy(pw1b_hbm, pw1b_ref, sem),
               pltpu.make_async_copy(pw2_hbm, pw2_ref, sem),
               pltpu.make_async_copy(pw3_hbm, pw3_ref, sem),
               pltpu.make_async_copy(fw1a_hbm, fw1a_ref, sem),
               pltpu.make_async_copy(fw1b_hbm, fw1b_ref, sem),
               pltpu.make_async_copy(fw2_hbm, fw2_ref, sem)]
        for c in cps:
            c.start()
        for c in cps:
            c.wait()

    pmatf = p_ref[:, 0, 0, :]
    pmat = pmatf.astype(_BF16)
    sp_ref[0] = jnp.dot(pmatf, psp_ref[0], preferred_element_type=_F32)
    opc = jnp.dot(pmat, ppc_ref[0], preferred_element_type=_F32)
    h = (jnp.dot(pc_ref[0], pw1a_ref[...], preferred_element_type=_F32)
         + jnp.dot(opc.astype(_BF16), pw1b_ref[...], preferred_element_type=_F32)
         + pb1_ref[...])
    h = jnp.maximum(h, 0.0).astype(_BF16)
    h = jnp.dot(h, pw2_ref[...], preferred_element_type=_F32) + pb2_ref[...]
    h = jnp.dot(h.astype(_BF16), pw3_ref[...], preferred_element_type=_F32)
    h = h + pb3_ref[...]
    hp = _layer_norm(h, plg_ref[...], plb_ref[...])

    h = (jnp.dot(hi_ref[0].astype(_BF16), fw1a_ref[...],
                 preferred_element_type=_F32)
         + jnp.dot(hp.astype(_BF16), fw1b_ref[...], preferred_element_type=_F32)
         + fb1_ref[...])
    h = jnp.maximum(h, 0.0).astype(_BF16)
    h = jnp.dot(h, fw2_ref[...], preferred_element_type=_F32) + fb2_ref[...]
    vis_ref[0] = _layer_norm(h, flg_ref[...], flb_ref[...])


def kernel(image_feature, point_cloud_feature, prev_image_feature,
           prev_point_cloud_feature, rel_dist_mask, prev_spatial,
           img_w1, img_b1, img_w2, img_b2, img_w3, img_b3, img_ln_g, img_ln_b,
           pc_w1, pc_b1, pc_w2, pc_b2, pc_w3, pc_b3, pc_ln_g, pc_ln_b,
           fu_w1, fu_b1, fu_w2, fu_b2, fu_ln_g, fu_ln_b):
    maskf = rel_dist_mask.astype(_F32)
    h = lambda a: a.astype(_BF16)
    img_h, pc_h = h(image_feature), h(point_cloud_feature)
    pimg_h, ppc_h = h(prev_image_feature), h(prev_point_cloud_feature)

    s_t = pl.pallas_call(
        _sim_body,
        grid=(_B,),
        in_specs=[
            pl.BlockSpec((1, _N, _IMG_D), lambda b: (b, 0, 0)),
            pl.BlockSpec((1, _N, _PC_D), lambda b: (b, 0, 0)),
            pl.BlockSpec((1, _N, _IMG_D), lambda b: (b, 0, 0)),
            pl.BlockSpec((1, _N, _PC_D), lambda b: (b, 0, 0)),
            pl.BlockSpec((1, _N, _N), lambda b: (b, 0, 0)),
        ],
        out_specs=pl.BlockSpec((_N, 1, 1, _N), lambda b: (0, b, 0, 0)),
        out_shape=jax.ShapeDtypeStruct((_N, _B, 1, _N), _F32),
    )(img_h, pc_h, pimg_h, ppc_h, maskf)

    p_t = pl.pallas_call(
        _match_body,
        out_shape=jax.ShapeDtypeStruct((_N, _B, 1, _N), _F32),
    )(s_t)

    full = lambda a: pl.BlockSpec(a.shape, lambda b: (0,) * a.ndim)
    bat = lambda d: pl.BlockSpec((1, _N, d), lambda b: (b, 0, 0))
    pspec = pl.BlockSpec((_N, 1, 1, _N), lambda b: (0, b, 0, 0))
    row = lambda a: a.reshape(1, -1)
    iw1a, iw1b = h(img_w1[:_IMG_D]), h(img_w1[_IMG_D:])
    pw1a, pw1b = h(pc_w1[:_PC_D]), h(pc_w1[_PC_D:])
    fw1a, fw1b = h(fu_w1[:_VIS_D]), h(fu_w1[_VIS_D:])
    iw2, iw3 = h(img_w2), h(img_w3)
    pw2, pw3 = h(pc_w2), h(pc_w3)
    fw2 = h(fu_w2)
    ib1, ib2, ib3 = row(img_b1), row(img_b2), row(img_b3)
    ilg, ilb = row(img_ln_g), row(img_ln_b)
    pb1, pb2, pb3 = row(pc_b1), row(pc_b2), row(pc_b3)
    plg, plb = row(pc_ln_g), row(pc_ln_b)
    fb1, fb2 = row(fu_b1), row(fu_b2)
    flg, flb = row(fu_ln_g), row(fu_ln_b)

    anyspec = pl.BlockSpec(memory_space=pl.ANY)
    hi = pl.pallas_call(
        _img_chain_body,
        grid=(_B,),
        in_specs=[pspec, bat(_IMG_D), bat(_IMG_D), anyspec, anyspec, anyspec,
                  anyspec, full(ib1), full(ib2), full(ib3),
                  full(ilg), full(ilb)],
        out_specs=bat(_VIS_D),
        out_shape=jax.ShapeDtypeStruct((_B, _N, _VIS_D), _F32),
        scratch_shapes=[
            pltpu.VMEM((_IMG_D, _IMG_D), _BF16),
            pltpu.VMEM((_IMG_D, _IMG_D), _BF16),
            pltpu.VMEM((_IMG_D, _IMG_D), _BF16),
            pltpu.VMEM((_IMG_D, _VIS_D), _BF16),
            pltpu.SemaphoreType.DMA,
        ],
    )(p_t, img_h, pimg_h, iw1a, iw1b, iw2, iw3, ib1, ib2, ib3, ilg, ilb)

    vis, new_sp = pl.pallas_call(
        _pcfu_body,
        grid=(_B,),
        in_specs=[pspec, bat(_PC_D), bat(_PC_D), bat(_SP_D), bat(_VIS_D),
                  anyspec, anyspec, anyspec, anyspec, anyspec, anyspec,
                  anyspec, full(pb1), full(pb2), full(pb3), full(plg),
                  full(plb), full(fb1), full(fb2), full(flg), full(flb)],
        out_specs=[bat(_VIS_D), bat(_SP_D)],
        out_shape=[
            jax.ShapeDtypeStruct((_B, _N, _VIS_D), _F32),
            jax.ShapeDtypeStruct((_B, _N, _SP_D), _F32),
        ],
        scratch_shapes=[
            pltpu.VMEM((_PC_D, _PC_D), _BF16),
            pltpu.VMEM((_PC_D, _PC_D), _BF16),
            pltpu.VMEM((_PC_D, _PC_D), _BF16),
            pltpu.VMEM((_PC_D, _VIS_D), _BF16),
            pltpu.VMEM((_VIS_D, _VIS_D), _BF16),
            pltpu.VMEM((_VIS_D, _VIS_D), _BF16),
            pltpu.VMEM((_VIS_D, _VIS_D), _BF16),
            pltpu.SemaphoreType.DMA,
        ],
    )(p_t, pc_h, ppc_h, prev_spatial, hi,
      pw1a, pw1b, pw2, pw3, fw1a, fw1b, fw2,
      pb1, pb2, pb3, plg, plb, fb1, fb2, flg, flb)

    return vis, new_sp


# speedup vs baseline: 12.7705x; 1.0385x over previous
import jax
import jax.numpy as jnp
from jax import lax
from jax.experimental import pallas as pl
from jax.experimental.pallas import tpu as pltpu

_B, _N = 16, 256
_IMG_D, _PC_D = 2048, 768
_VIS_D, _SP_D = 768, 8
_THR = 0.5
_EPS = 1e-8
_NEG = -1e30
_F32 = jnp.float32
_BPG = 2
_BPG2 = 4
_BF16 = jnp.bfloat16


def _sim_body(img_ref, pc_ref, pimg_ref, ppc_ref, mask_ref, s_ref):
    img = img_ref[0]
    pc = pc_ref[0]
    pimg = pimg_ref[0]
    ppc = ppc_ref[0]
    dn = (((1,), (1,)), ((), ()))
    dot = lax.dot_general(img, pimg, dn, preferred_element_type=_F32)
    dot += lax.dot_general(pc, ppc, dn, preferred_element_type=_F32)
    sq = lambda a: jnp.sum(a.astype(_F32) ** 2, 1, keepdims=True)
    nf = jnp.maximum(jnp.sqrt(sq(img) + sq(pc)), _EPS)
    npr = jnp.maximum(jnp.sqrt(sq(pimg) + sq(ppc)), _EPS)
    sim = dot / (nf * npr.reshape(1, _N))
    s_ref[:, 0, 0, :] = jnp.where(mask_ref[0] != 0, sim, _NEG)


def _match_body(s_ref, p_ref):
    def step(i, visited):
        row = s_ref[i, :, 0, :]
        s = jnp.where(visited != 0, _NEG, row)
        m = jnp.max(s, axis=1, keepdims=True)
        iota = lax.broadcasted_iota(jnp.int32, (_B, _N), 1)
        cand = jnp.where(s == m, iota, _N)
        j = jnp.min(cand, axis=1, keepdims=True)
        hit = (iota == j) & (m >= _THR)
        p_ref[i, :, 0, :] = hit.astype(_F32)
        return visited | hit.astype(jnp.int32)

    def slow():
        lax.fori_loop(0, _N, step, jnp.zeros((_B, _N), jnp.int32))

    def fast():
        p_ref[...] = jnp.zeros((_N, _B, 1, _N), _F32)

    lax.cond(jnp.max(s_ref[...]) >= _THR, slow, fast)


def _layer_norm(x, g, b):
    m = jnp.mean(x, axis=1, keepdims=True)
    v = jnp.mean((x - m) ** 2, axis=1, keepdims=True)
    return (x - m) / jnp.sqrt(v + 1e-5) * g + b


def _img_chain_body(p_ref, img_ref, pimg_ref, w1a_hbm, w1b_hbm, w2_hbm,
                    w3_hbm, b1_ref, b2_ref, b3_ref, g_ref, bb_ref, hi_ref,
                    w1a_ref, w1b_ref, w2_ref, w3_ref, sem):
    @pl.when(pl.program_id(0) == 0)
    def _load_weights():
        cps = [pltpu.make_async_copy(w1a_hbm, w1a_ref, sem),
               pltpu.make_async_copy(w1b_hbm, w1b_ref, sem),
               pltpu.make_async_copy(w2_hbm, w2_ref, sem),
               pltpu.make_async_copy(w3_hbm, w3_ref, sem)]
        for c in cps:
            c.start()
        for c in cps:
            c.wait()

    oimg = jnp.concatenate(
        [jnp.dot(p_ref[:, k, 0, :].astype(_BF16), pimg_ref[k],
                 preferred_element_type=_F32) for k in range(_BPG)], axis=0)
    x = img_ref[...].reshape(_BPG * _N, _IMG_D)
    h = (jnp.dot(x, w1a_ref[...], preferred_element_type=_F32)
         + jnp.dot(oimg.astype(_BF16), w1b_ref[...], preferred_element_type=_F32)
         + b1_ref[...])
    h = jnp.maximum(h, 0.0).astype(_BF16)
    h = jnp.dot(h, w2_ref[...], preferred_element_type=_F32) + b2_ref[...]
    h = jnp.dot(h.astype(_BF16), w3_ref[...], preferred_element_type=_F32)
    h = h + b3_ref[...]
    hi_ref[...] = _layer_norm(h, g_ref[...], bb_ref[...]).reshape(
        _BPG, _N, _VIS_D)


def _pcfu_body(p_ref, pc_ref, ppc_ref, psp_ref, hi_ref,
               pw1a_hbm, pw1b_hbm, pw2_hbm, pw3_hbm,
               fw1a_hbm, fw1b_hbm, fw2_hbm,
               pb1_ref, pb2_ref, pb3_ref, plg_ref, plb_ref,
               fb1_ref, fb2_ref, flg_ref, flb_ref, vis_ref, sp_ref,
               pw1a_ref, pw1b_ref, pw2_ref, pw3_ref,
               fw1a_ref, fw1b_ref, fw2_ref, sem):
    @pl.when(pl.program_id(0) == 0)
    def _load_weights():
        cps = [pltpu.make_async_copy(pw1a_hbm, pw1a_ref, sem),
               pltpu.make_async_copy(pw1b_hbm, pw1b_ref, sem),
               pltpu.make_async_copy(pw2_hbm, pw2_ref, sem),
               pltpu.make_async_copy(pw3_hbm, pw3_ref, sem),
               pltpu.make_async_copy(fw1a_hbm, fw1a_ref, sem),
               pltpu.make_async_copy(fw1b_hbm, fw1b_ref, sem),
               pltpu.make_async_copy(fw2_hbm, fw2_ref, sem)]
        for c in cps:
            c.start()
        for c in cps:
            c.wait()

    for k in range(_BPG2):
        sp_ref[k] = jnp.dot(p_ref[:, k, 0, :], psp_ref[k],
                            preferred_element_type=_F32)
    opc = jnp.concatenate(
        [jnp.dot(p_ref[:, k, 0, :].astype(_BF16), ppc_ref[k],
                 preferred_element_type=_F32) for k in range(_BPG2)], axis=0)
    xpc = pc_ref[...].reshape(_BPG2 * _N, _PC_D)
    h = (jnp.dot(xpc, pw1a_ref[...], preferred_element_type=_F32)
         + jnp.dot(opc.astype(_BF16), pw1b_ref[...], preferred_element_type=_F32)
         + pb1_ref[...])
    h = jnp.maximum(h, 0.0).astype(_BF16)
    h = jnp.dot(h, pw2_ref[...], preferred_element_type=_F32) + pb2_ref[...]
    h = jnp.dot(h.astype(_BF16), pw3_ref[...], preferred_element_type=_F32)
    h = h + pb3_ref[...]
    hp = _layer_norm(h, plg_ref[...], plb_ref[...])

    h = (jnp.dot(hi_ref[...].reshape(_BPG2 * _N, _VIS_D).astype(_BF16),
                 fw1a_ref[...], preferred_element_type=_F32)
         + jnp.dot(hp.astype(_BF16), fw1b_ref[...], preferred_element_type=_F32)
         + fb1_ref[...])
    h = jnp.maximum(h, 0.0).astype(_BF16)
    h = jnp.dot(h, fw2_ref[...], preferred_element_type=_F32) + fb2_ref[...]
    vis_ref[...] = _layer_norm(h, flg_ref[...], flb_ref[...]).reshape(
        _BPG2, _N, _VIS_D)


def kernel(image_feature, point_cloud_feature, prev_image_feature,
           prev_point_cloud_feature, rel_dist_mask, prev_spatial,
           img_w1, img_b1, img_w2, img_b2, img_w3, img_b3, img_ln_g, img_ln_b,
           pc_w1, pc_b1, pc_w2, pc_b2, pc_w3, pc_b3, pc_ln_g, pc_ln_b,
           fu_w1, fu_b1, fu_w2, fu_b2, fu_ln_g, fu_ln_b):
    maskf = rel_dist_mask.astype(_F32)
    h = lambda a: a.astype(_BF16)
    img_h, pc_h = h(image_feature), h(point_cloud_feature)
    pimg_h, ppc_h = h(prev_image_feature), h(prev_point_cloud_feature)

    s_t = pl.pallas_call(
        _sim_body,
        grid=(_B,),
        in_specs=[
            pl.BlockSpec((1, _N, _IMG_D), lambda b: (b, 0, 0)),
            pl.BlockSpec((1, _N, _PC_D), lambda b: (b, 0, 0)),
            pl.BlockSpec((1, _N, _IMG_D), lambda b: (b, 0, 0)),
            pl.BlockSpec((1, _N, _PC_D), lambda b: (b, 0, 0)),
            pl.BlockSpec((1, _N, _N), lambda b: (b, 0, 0)),
        ],
        out_specs=pl.BlockSpec((_N, 1, 1, _N), lambda b: (0, b, 0, 0)),
        out_shape=jax.ShapeDtypeStruct((_N, _B, 1, _N), _F32),
    )(img_h, pc_h, pimg_h, ppc_h, maskf)

    p_t = pl.pallas_call(
        _match_body,
        out_shape=jax.ShapeDtypeStruct((_N, _B, 1, _N), _F32),
    )(s_t)

    full = lambda a: pl.BlockSpec(a.shape, lambda b: (0,) * a.ndim)
    bat = lambda d: pl.BlockSpec((_BPG, _N, d), lambda b: (b, 0, 0))
    pspec = pl.BlockSpec((_N, _BPG, 1, _N), lambda b: (0, b, 0, 0))
    bat2 = lambda d: pl.BlockSpec((_BPG2, _N, d), lambda b: (b, 0, 0))
    pspec2 = pl.BlockSpec((_N, _BPG2, 1, _N), lambda b: (0, b, 0, 0))
    row = lambda a: a.reshape(1, -1)
    iw1a, iw1b = h(img_w1[:_IMG_D]), h(img_w1[_IMG_D:])
    pw1a, pw1b = h(pc_w1[:_PC_D]), h(pc_w1[_PC_D:])
    fw1a, fw1b = h(fu_w1[:_VIS_D]), h(fu_w1[_VIS_D:])
    iw2, iw3 = h(img_w2), h(img_w3)
    pw2, pw3 = h(pc_w2), h(pc_w3)
    fw2 = h(fu_w2)
    ib1, ib2, ib3 = row(img_b1), row(img_b2), row(img_b3)
    ilg, ilb = row(img_ln_g), row(img_ln_b)
    pb1, pb2, pb3 = row(pc_b1), row(pc_b2), row(pc_b3)
    plg, plb = row(pc_ln_g), row(pc_ln_b)
    fb1, fb2 = row(fu_b1), row(fu_b2)
    flg, flb = row(fu_ln_g), row(fu_ln_b)

    anyspec = pl.BlockSpec(memory_space=pl.ANY)
    hi = pl.pallas_call(
        _img_chain_body,
        grid=(_B // _BPG,),
        in_specs=[pspec, bat(_IMG_D), bat(_IMG_D), anyspec, anyspec, anyspec,
                  anyspec, full(ib1), full(ib2), full(ib3),
                  full(ilg), full(ilb)],
        out_specs=bat(_VIS_D),
        out_shape=jax.ShapeDtypeStruct((_B, _N, _VIS_D), _F32),
        scratch_shapes=[
            pltpu.VMEM((_IMG_D, _IMG_D), _BF16),
            pltpu.VMEM((_IMG_D, _IMG_D), _BF16),
            pltpu.VMEM((_IMG_D, _IMG_D), _BF16),
            pltpu.VMEM((_IMG_D, _VIS_D), _BF16),
            pltpu.SemaphoreType.DMA,
        ],
    )(p_t, img_h, pimg_h, iw1a, iw1b, iw2, iw3, ib1, ib2, ib3, ilg, ilb)

    vis, new_sp = pl.pallas_call(
        _pcfu_body,
        grid=(_B // _BPG2,),
        in_specs=[pspec2, bat2(_PC_D), bat2(_PC_D), bat2(_SP_D), bat2(_VIS_D),
                  anyspec, anyspec, anyspec, anyspec, anyspec, anyspec,
                  anyspec, full(pb1), full(pb2), full(pb3), full(plg),
                  full(plb), full(fb1), full(fb2), full(flg), full(flb)],
        out_specs=[bat2(_VIS_D), bat2(_SP_D)],
        out_shape=[
            jax.ShapeDtypeStruct((_B, _N, _VIS_D), _F32),
            jax.ShapeDtypeStruct((_B, _N, _SP_D), _F32),
        ],
        scratch_shapes=[
            pltpu.VMEM((_PC_D, _PC_D), _BF16),
            pltpu.VMEM((_PC_D, _PC_D), _BF16),
            pltpu.VMEM((_PC_D, _PC_D), _BF16),
            pltpu.VMEM((_PC_D, _VIS_D), _BF16),
            pltpu.VMEM((_VIS_D, _VIS_D), _BF16),
            pltpu.VMEM((_VIS_D, _VIS_D), _BF16),
            pltpu.VMEM((_VIS_D, _VIS_D), _BF16),
            pltpu.SemaphoreType.DMA,
        ],
    )(p_t, pc_h, ppc_h, prev_spatial, hi,
      pw1a, pw1b, pw2, pw3, fw1a, fw1b, fw2,
      pb1, pb2, pb3, plg, plb, fb1, fb2, flg, flb)

    return vis, new_sp


# speedup vs baseline: 14.2171x; 1.1133x over previous
import jax
import jax.numpy as jnp
from jax import lax
from jax.experimental import pallas as pl
from jax.experimental.pallas import tpu as pltpu

_B, _N = 16, 256
_IMG_D, _PC_D = 2048, 768
_VIS_D, _SP_D = 768, 8
_THR = 0.5
_EPS = 1e-8
_NEG = -1e30
_F32 = jnp.float32
_BPG = 2
_BPG2 = 4
_BF16 = jnp.bfloat16


def _sim_body(img_ref, pc_ref, pimg_ref, ppc_ref, mask_ref, s_ref):
    img = img_ref[0]
    pc = pc_ref[0]
    pimg = pimg_ref[0]
    ppc = ppc_ref[0]
    dn = (((1,), (1,)), ((), ()))
    dot = lax.dot_general(img, pimg, dn, preferred_element_type=_F32)
    dot += lax.dot_general(pc, ppc, dn, preferred_element_type=_F32)
    sq = lambda a: jnp.sum(a.astype(_F32) ** 2, 1, keepdims=True)
    nf = jnp.maximum(jnp.sqrt(sq(img) + sq(pc)), _EPS)
    npr = jnp.maximum(jnp.sqrt(sq(pimg) + sq(ppc)), _EPS)
    sim = dot / (nf * npr.reshape(1, _N))
    s_ref[:, 0, 0, :] = jnp.where(mask_ref[0] != 0, sim, _NEG)


def _match_body(s_ref, p_ref, flag_ref):
    def step(i, visited):
        row = s_ref[i, :, 0, :]
        s = jnp.where(visited != 0, _NEG, row)
        m = jnp.max(s, axis=1, keepdims=True)
        iota = lax.broadcasted_iota(jnp.int32, (_B, _N), 1)
        cand = jnp.where(s == m, iota, _N)
        j = jnp.min(cand, axis=1, keepdims=True)
        hit = (iota == j) & (m >= _THR)
        p_ref[i, :, 0, :] = hit.astype(_F32)
        return visited | hit.astype(jnp.int32)

    def slow():
        lax.fori_loop(0, _N, step, jnp.zeros((_B, _N), jnp.int32))

    def fast():
        p_ref[...] = jnp.zeros((_N, _B, 1, _N), _F32)

    any_hit = jnp.max(s_ref[...]) >= _THR
    flag_ref[0, 0] = any_hit.astype(jnp.int32)
    lax.cond(any_hit, slow, fast)


def _layer_norm(x, g, b):
    m = jnp.mean(x, axis=1, keepdims=True)
    v = jnp.mean((x - m) ** 2, axis=1, keepdims=True)
    return (x - m) / jnp.sqrt(v + 1e-5) * g + b


def _img_chain_body(flag_ref, p_ref, img_ref, pimg_ref, w1a_hbm, w1b_hbm,
                    w2_hbm, w3_hbm, b1_ref, b2_ref, b3_ref, g_ref, bb_ref,
                    hi_ref, w1a_ref, w1b_ref, w2_ref, w3_ref, sem):
    @pl.when(pl.program_id(0) == 0)
    def _load_weights():
        cps = [pltpu.make_async_copy(w1a_hbm, w1a_ref, sem),
               pltpu.make_async_copy(w1b_hbm, w1b_ref, sem),
               pltpu.make_async_copy(w2_hbm, w2_ref, sem),
               pltpu.make_async_copy(w3_hbm, w3_ref, sem)]
        for c in cps:
            c.start()
        for c in cps:
            c.wait()

    x = img_ref[...].reshape(_BPG * _N, _IMG_D)
    base = jnp.dot(x, w1a_ref[...], preferred_element_type=_F32) + b1_ref[...]

    def _with_prev():
        oimg = jnp.concatenate(
            [jnp.dot(p_ref[:, k, 0, :].astype(_BF16), pimg_ref[k],
                     preferred_element_type=_F32) for k in range(_BPG)], axis=0)
        return base + jnp.dot(oimg.astype(_BF16), w1b_ref[...],
                              preferred_element_type=_F32)

    h = lax.cond(flag_ref[0, 0] == 1, _with_prev, lambda: base)
    h = jnp.maximum(h, 0.0).astype(_BF16)
    h = jnp.dot(h, w2_ref[...], preferred_element_type=_F32) + b2_ref[...]
    h = jnp.dot(h.astype(_BF16), w3_ref[...], preferred_element_type=_F32)
    h = h + b3_ref[...]
    hi_ref[...] = _layer_norm(h, g_ref[...], bb_ref[...]).reshape(
        _BPG, _N, _VIS_D)


def _pcfu_body(flag_ref, p_ref, pc_ref, ppc_ref, psp_ref, hi_ref,
               pw1a_hbm, pw1b_hbm, pw2_hbm, pw3_hbm,
               fw1a_hbm, fw1b_hbm, fw2_hbm,
               pb1_ref, pb2_ref, pb3_ref, plg_ref, plb_ref,
               fb1_ref, fb2_ref, flg_ref, flb_ref, vis_ref, sp_ref,
               pw1a_ref, pw1b_ref, pw2_ref, pw3_ref,
               fw1a_ref, fw1b_ref, fw2_ref, sem):
    @pl.when(pl.program_id(0) == 0)
    def _load_weights():
        cps = [pltpu.make_async_copy(pw1a_hbm, pw1a_ref, sem),
               pltpu.make_async_copy(pw1b_hbm, pw1b_ref, sem),
               pltpu.make_async_copy(pw2_hbm, pw2_ref, sem),
               pltpu.make_async_copy(pw3_hbm, pw3_ref, sem),
               pltpu.make_async_copy(fw1a_hbm, fw1a_ref, sem),
               pltpu.make_async_copy(fw1b_hbm, fw1b_ref, sem),
               pltpu.make_async_copy(fw2_hbm, fw2_ref, sem)]
        for c in cps:
            c.start()
        for c in cps:
            c.wait()

    matched = flag_ref[0, 0] == 1

    @pl.when(matched)
    def _sp_gather():
        for k in range(_BPG2):
            sp_ref[k] = jnp.dot(p_ref[:, k, 0, :], psp_ref[k],
                                preferred_element_type=_F32)

    @pl.when(jnp.logical_not(matched))
    def _sp_zero():
        sp_ref[...] = jnp.zeros((_BPG2, _N, _SP_D), _F32)

    xpc = pc_ref[...].reshape(_BPG2 * _N, _PC_D)
    pbase = jnp.dot(xpc, pw1a_ref[...], preferred_element_type=_F32) + pb1_ref[...]

    def _with_prev():
        opc = jnp.concatenate(
            [jnp.dot(p_ref[:, k, 0, :].astype(_BF16), ppc_ref[k],
                     preferred_element_type=_F32) for k in range(_BPG2)], axis=0)
        return pbase + jnp.dot(opc.astype(_BF16), pw1b_ref[...],
                               preferred_element_type=_F32)

    h = lax.cond(matched, _with_prev, lambda: pbase)
    h = jnp.maximum(h, 0.0).astype(_BF16)
    h = jnp.dot(h, pw2_ref[...], preferred_element_type=_F32) + pb2_ref[...]
    h = jnp.dot(h.astype(_BF16), pw3_ref[...], preferred_element_type=_F32)
    h = h + pb3_ref[...]
    hp = _layer_norm(h, plg_ref[...], plb_ref[...])

    h = (jnp.dot(hi_ref[...].reshape(_BPG2 * _N, _VIS_D).astype(_BF16),
                 fw1a_ref[...], preferred_element_type=_F32)
         + jnp.dot(hp.astype(_BF16), fw1b_ref[...], preferred_element_type=_F32)
         + fb1_ref[...])
    h = jnp.maximum(h, 0.0).astype(_BF16)
    h = jnp.dot(h, fw2_ref[...], preferred_element_type=_F32) + fb2_ref[...]
    vis_ref[...] = _layer_norm(h, flg_ref[...], flb_ref[...]).reshape(
        _BPG2, _N, _VIS_D)


def kernel(image_feature, point_cloud_feature, prev_image_feature,
           prev_point_cloud_feature, rel_dist_mask, prev_spatial,
           img_w1, img_b1, img_w2, img_b2, img_w3, img_b3, img_ln_g, img_ln_b,
           pc_w1, pc_b1, pc_w2, pc_b2, pc_w3, pc_b3, pc_ln_g, pc_ln_b,
           fu_w1, fu_b1, fu_w2, fu_b2, fu_ln_g, fu_ln_b):
    maskf = rel_dist_mask.astype(_F32)
    h = lambda a: a.astype(_BF16)
    img_h, pc_h = h(image_feature), h(point_cloud_feature)
    pimg_h, ppc_h = h(prev_image_feature), h(prev_point_cloud_feature)

    s_t = pl.pallas_call(
        _sim_body,
        grid=(_B,),
        in_specs=[
            pl.BlockSpec((1, _N, _IMG_D), lambda b: (b, 0, 0)),
            pl.BlockSpec((1, _N, _PC_D), lambda b: (b, 0, 0)),
            pl.BlockSpec((1, _N, _IMG_D), lambda b: (b, 0, 0)),
            pl.BlockSpec((1, _N, _PC_D), lambda b: (b, 0, 0)),
            pl.BlockSpec((1, _N, _N), lambda b: (b, 0, 0)),
        ],
        out_specs=pl.BlockSpec((_N, 1, 1, _N), lambda b: (0, b, 0, 0)),
        out_shape=jax.ShapeDtypeStruct((_N, _B, 1, _N), _F32),
    )(img_h, pc_h, pimg_h, ppc_h, maskf)

    p_t, hit_flag = pl.pallas_call(
        _match_body,
        out_specs=[pl.BlockSpec(memory_space=pltpu.VMEM),
                   pl.BlockSpec(memory_space=pltpu.SMEM)],
        out_shape=[jax.ShapeDtypeStruct((_N, _B, 1, _N), _F32),
                   jax.ShapeDtypeStruct((1, 1), jnp.int32)],
    )(s_t)

    full = lambda a: pl.BlockSpec(a.shape, lambda b: (0,) * a.ndim)
    bat = lambda d: pl.BlockSpec((_BPG, _N, d), lambda b: (b, 0, 0))
    pspec = pl.BlockSpec((_N, _BPG, 1, _N), lambda b: (0, b, 0, 0))
    bat2 = lambda d: pl.BlockSpec((_BPG2, _N, d), lambda b: (b, 0, 0))
    pspec2 = pl.BlockSpec((_N, _BPG2, 1, _N), lambda b: (0, b, 0, 0))
    row = lambda a: a.reshape(1, -1)
    iw1a, iw1b = h(img_w1[:_IMG_D]), h(img_w1[_IMG_D:])
    pw1a, pw1b = h(pc_w1[:_PC_D]), h(pc_w1[_PC_D:])
    fw1a, fw1b = h(fu_w1[:_VIS_D]), h(fu_w1[_VIS_D:])
    iw2, iw3 = h(img_w2), h(img_w3)
    pw2, pw3 = h(pc_w2), h(pc_w3)
    fw2 = h(fu_w2)
    ib1, ib2, ib3 = row(img_b1), row(img_b2), row(img_b3)
    ilg, ilb = row(img_ln_g), row(img_ln_b)
    pb1, pb2, pb3 = row(pc_b1), row(pc_b2), row(pc_b3)
    plg, plb = row(pc_ln_g), row(pc_ln_b)
    fb1, fb2 = row(fu_b1), row(fu_b2)
    flg, flb = row(fu_ln_g), row(fu_ln_b)

    anyspec = pl.BlockSpec(memory_space=pl.ANY)
    hi = pl.pallas_call(
        _img_chain_body,
        grid=(_B // _BPG,),
        in_specs=[pl.BlockSpec(memory_space=pltpu.SMEM), pspec, bat(_IMG_D),
                  bat(_IMG_D), anyspec, anyspec, anyspec,
                  anyspec, full(ib1), full(ib2), full(ib3),
                  full(ilg), full(ilb)],
        out_specs=bat(_VIS_D),
        out_shape=jax.ShapeDtypeStruct((_B, _N, _VIS_D), _F32),
        scratch_shapes=[
            pltpu.VMEM((_IMG_D, _IMG_D), _BF16),
            pltpu.VMEM((_IMG_D, _IMG_D), _BF16),
            pltpu.VMEM((_IMG_D, _IMG_D), _BF16),
            pltpu.VMEM((_IMG_D, _VIS_D), _BF16),
            pltpu.SemaphoreType.DMA,
        ],
    )(hit_flag, p_t, img_h, pimg_h, iw1a, iw1b, iw2, iw3, ib1, ib2, ib3,
      ilg, ilb)

    vis, new_sp = pl.pallas_call(
        _pcfu_body,
        grid=(_B // _BPG2,),
        in_specs=[pl.BlockSpec(memory_space=pltpu.SMEM), pspec2,
                  bat2(_PC_D), bat2(_PC_D), bat2(_SP_D), bat2(_VIS_D),
                  anyspec, anyspec, anyspec, anyspec, anyspec, anyspec,
                  anyspec, full(pb1), full(pb2), full(pb3), full(plg),
                  full(plb), full(fb1), full(fb2), full(flg), full(flb)],
        out_specs=[bat2(_VIS_D), bat2(_SP_D)],
        out_shape=[
            jax.ShapeDtypeStruct((_B, _N, _VIS_D), _F32),
            jax.ShapeDtypeStruct((_B, _N, _SP_D), _F32),
        ],
        scratch_shapes=[
            pltpu.VMEM((_PC_D, _PC_D), _BF16),
            pltpu.VMEM((_PC_D, _PC_D), _BF16),
            pltpu.VMEM((_PC_D, _PC_D), _BF16),
            pltpu.VMEM((_PC_D, _VIS_D), _BF16),
            pltpu.VMEM((_VIS_D, _VIS_D), _BF16),
            pltpu.VMEM((_VIS_D, _VIS_D), _BF16),
            pltpu.VMEM((_VIS_D, _VIS_D), _BF16),
            pltpu.SemaphoreType.DMA,
        ],
    )(hit_flag, p_t, pc_h, ppc_h, prev_spatial, hi,
      pw1a, pw1b, pw2, pw3, fw1a, fw1b, fw2,
      pb1, pb2, pb3, plg, plb, fb1, fb2, flg, flb)

    return vis, new_sp
